# baseline (speedup 1.0000x reference)
import functools

import jax
import jax.numpy as jnp
from jax import lax
from jax.experimental import pallas as pl
from jax.experimental.pallas import tpu as pltpu
from jax.experimental.pallas import tpu_sc as plsc

N = 2048
ROWS = 256
NC, NS, L = 2, 16, 16
NW = NC * NS
RPW = ROWS // NW
EPS = 1e-8
DT = 1e-3
GROUPS = N // L
BS_STEPS = 12


def _prep_body(x_ref, y_ref, xt_ref, yt_ref):
    m = jnp.minimum(jnp.min(x_ref[...]), jnp.min(y_ref[...]))
    xt_ref[...] = jnp.transpose(x_ref[...], (0, 2, 1)) - m + EPS
    yt_ref[...] = jnp.transpose(y_ref[...], (0, 2, 1)) - m + EPS


_DNUMS = lax.GatherDimensionNumbers(
    offset_dims=(), collapsed_slice_dims=(0,), start_index_map=(0,)
)


def _vgather(v, idx):
    return lax.gather(
        v, idx[:, None], _DNUMS, slice_sizes=(1,),
        mode=lax.GatherScatterMode.PROMISE_IN_BOUNDS,
    )


def _bcast_last(v):
    return _vgather(v, jnp.full((L,), L - 1, jnp.int32))


def _cumsum_inplace(ref):
    def chunk2(i, carry):
        c0 = ref[pl.ds((2 * i) * L, L)]
        c1 = ref[pl.ds((2 * i + 1) * L, L)]
        s0 = plsc.cumsum(c0)
        s1 = plsc.cumsum(c1)
        cs0 = s0 + carry
        cs1 = s1 + _bcast_last(cs0)
        ref[pl.ds((2 * i) * L, L)] = cs0
        ref[pl.ds((2 * i + 1) * L, L)] = cs1
        return _bcast_last(cs1)
    return lax.fori_loop(0, GROUPS // 2, chunk2, jnp.zeros((L,), jnp.float32))


_SHIFT_IDX = None


def _direction(q_ref, t_ref, r_qt, r_tq, right):
    lanes = lax.iota(jnp.int32, L)
    shift_idx = jnp.maximum(lanes - 1, 0)

    def finish(g, q, c, carry_q):
        tprev = plsc.load_gather(t_ref, [jnp.maximum(c - 1, 0)])
        tprev = jnp.where(c > 0, tprev * r_tq, 0.0)
        qprev = _vgather(q, shift_idx)
        qprev = jnp.where(lanes == 0, carry_q, qprev)
        ivec = g * L + lanes
        delta = q - jnp.maximum(qprev, tprev)
        di = (ivec - jnp.minimum(c, N - 1)).astype(jnp.float32)
        return delta * di * di, _bcast_last(q)

    def group2(i, state):
        acc0, acc1, carry_q = state
        g0 = 2 * i
        g1 = 2 * i + 1
        q0 = q_ref[pl.ds(g0 * L, L)]
        q1 = q_ref[pl.ds(g1 * L, L)]
        qs0 = q0 * r_qt
        qs1 = q1 * r_qt
        zi = jnp.zeros((L,), jnp.int32)
        nv = jnp.full((L,), N, jnp.int32)

        def step(_, st):
            lo0, hi0, lo1, hi1 = st
            mid0 = lax.shift_right_logical(lo0 + hi0, 1)
            mid1 = lax.shift_right_logical(lo1 + hi1, 1)
            tv0 = plsc.load_gather(t_ref, [mid0])
            tv1 = plsc.load_gather(t_ref, [mid1])
            p0 = (tv0 <= qs0) if right else (tv0 < qs0)
            p1 = (tv1 <= qs1) if right else (tv1 < qs1)
            return (jnp.where(p0, mid0 + 1, lo0), jnp.where(p0, hi0, mid0),
                    jnp.where(p1, mid1 + 1, lo1), jnp.where(p1, hi1, mid1))

        c0, _, c1, _ = lax.fori_loop(0, BS_STEPS, step, (zi, nv, zi, nv))
        w0, carry_q = finish(g0, q0, c0, carry_q)
        w1, carry_q = finish(g1, q1, c1, carry_q)
        return acc0 + w0, acc1 + w1, carry_q

    z = jnp.zeros((L,), jnp.float32)
    acc0, acc1, _ = lax.fori_loop(0, GROUPS // 2, group2, (z, z, z))
    return acc0 + acc1


def _sc_body(xt_hbm, yt_hbm, out_hbm, a0_v, b0_v, a1_v, b1_v, w_v, sem0, sem1):
    wid = lax.axis_index("s") * NC + lax.axis_index("c")
    lanes = lax.iota(jnp.int32, L)
    base = wid * RPW

    def start(r, a_v, b_v, sem):
        pltpu.async_copy(
            xt_hbm.at[pl.ds((base + r) * N, N)], a_v.at[pl.ds(0, N)], sem)
        pltpu.async_copy(
            yt_hbm.at[pl.ds((base + r) * N, N)], b_v.at[pl.ds(0, N)], sem)

    def wait(r, a_v, b_v, sem):
        pltpu.make_async_copy(
            xt_hbm.at[pl.ds((base + r) * N, N)], a_v.at[pl.ds(0, N)], sem
        ).wait()
        pltpu.make_async_copy(
            yt_hbm.at[pl.ds((base + r) * N, N)], b_v.at[pl.ds(0, N)], sem
        ).wait()

    def compute(r, a_v, b_v, wvec):
        ta_v = _cumsum_inplace(a_v)
        tb_v = _cumsum_inplace(b_v)
        inf = jnp.full((L,), jnp.inf, jnp.float32)
        a_v[pl.ds(N, L)] = inf
        b_v[pl.ds(N, L)] = inf
        r_ab = tb_v / ta_v
        r_ba = ta_v / tb_v
        acc = (_direction(a_v, b_v, r_ab, r_ba, right=False) / ta_v
               + _direction(b_v, a_v, r_ba, r_ab, right=True) / tb_v)
        w = jnp.sum(acc) * (DT * DT)
        return jnp.where(lanes == r, w, wvec)

    start(0, a0_v, b0_v, sem0)

    def row_pair(i, wvec):
        r0 = 2 * i
        r1 = 2 * i + 1
        start(r1, a1_v, b1_v, sem1)
        wait(r0, a0_v, b0_v, sem0)
        wvec = compute(r0, a0_v, b0_v, wvec)

        @pl.when(i < RPW // 2 - 1)
        def _():
            start(r0 + 2, a0_v, b0_v, sem0)

        wait(r1, a1_v, b1_v, sem1)
        return compute(r1, a1_v, b1_v, wvec)

    wvec = lax.fori_loop(0, RPW // 2, row_pair, jnp.zeros((L,), jnp.float32))
    w_v[...] = wvec
    pltpu.sync_copy(w_v, out_hbm.at[wid])


def _sum_body(p_ref, o_ref):
    o_ref[...] = jnp.full((1, 1), jnp.sum(p_ref[...]), jnp.float32)


@jax.jit
def kernel(x, y):
    nb, nt, nr, nc = x.shape
    x2 = x.reshape(nb, nt, nr * nc)
    y2 = y.reshape(nb, nt, nr * nc)

    xt, yt = pl.pallas_call(
        _prep_body,
        out_shape=[
            jax.ShapeDtypeStruct((nb, nr * nc, nt), jnp.float32),
            jax.ShapeDtypeStruct((nb, nr * nc, nt), jnp.float32),
        ],
    )(x2, y2)

    mesh = plsc.VectorSubcoreMesh(
        core_axis_name="c", subcore_axis_name="s", num_cores=NC, num_subcores=NS
    )
    partials = pl.kernel(
        _sc_body,
        out_type=jax.ShapeDtypeStruct((NW, L), jnp.float32),
        mesh=mesh,
        compiler_params=pltpu.CompilerParams(needs_layout_passes=False),
        scratch_types=[
            pltpu.VMEM((N + L,), jnp.float32),
            pltpu.VMEM((N + L,), jnp.float32),
            pltpu.VMEM((N + L,), jnp.float32),
            pltpu.VMEM((N + L,), jnp.float32),
            pltpu.VMEM((L,), jnp.float32),
            pltpu.SemaphoreType.DMA,
            pltpu.SemaphoreType.DMA,
        ],
    )(xt.reshape(ROWS * N), yt.reshape(ROWS * N))

    loss = pl.pallas_call(
        _sum_body,
        out_shape=jax.ShapeDtypeStruct((1, 1), jnp.float32),
    )(partials.reshape(4, 128))
    return loss[0, 0]


# speedup vs baseline: 1.7710x; 1.7710x over previous
import functools

import jax
import jax.numpy as jnp
from jax import lax
from jax.experimental import pallas as pl
from jax.experimental.pallas import tpu as pltpu
from jax.experimental.pallas import tpu_sc as plsc

N = 2048
ROWS = 256
NC, NS, L = 2, 16, 16
NW = NC * NS
RPW = ROWS // NW
EPS = 1e-8
DT = 1e-3
GROUPS = N // L
BS_STEPS = 12


def _prep_body(x_ref, y_ref, xt_ref, yt_ref):
    m = jnp.minimum(jnp.min(x_ref[...]), jnp.min(y_ref[...]))
    xt_ref[...] = jnp.transpose(x_ref[...], (0, 2, 1)) - m + EPS
    yt_ref[...] = jnp.transpose(y_ref[...], (0, 2, 1)) - m + EPS


_DNUMS = lax.GatherDimensionNumbers(
    offset_dims=(), collapsed_slice_dims=(0,), start_index_map=(0,)
)


def _vgather(v, idx):
    return lax.gather(
        v, idx[:, None], _DNUMS, slice_sizes=(1,),
        mode=lax.GatherScatterMode.PROMISE_IN_BOUNDS,
    )


def _bcast_last(v):
    return _vgather(v, jnp.full((L,), L - 1, jnp.int32))


def _cumsum_inplace(ref):
    def chunk2(i, carry):
        c0 = ref[pl.ds((2 * i) * L, L)]
        c1 = ref[pl.ds((2 * i + 1) * L, L)]
        s0 = plsc.cumsum(c0)
        s1 = plsc.cumsum(c1)
        cs0 = s0 + carry
        cs1 = s1 + _bcast_last(cs0)
        ref[pl.ds((2 * i) * L, L)] = cs0
        ref[pl.ds((2 * i + 1) * L, L)] = cs1
        return _bcast_last(cs1)
    return lax.fori_loop(0, GROUPS // 2, chunk2, jnp.zeros((L,), jnp.float32))


_SHIFT_IDX = None


def _direction(q_ref, t_ref, r_qt, r_tq, right):
    lanes = lax.iota(jnp.int32, L)
    shift_idx = jnp.maximum(lanes - 1, 0)

    def one_group(g, carry_q):
        q = q_ref[pl.ds(g * L, L)]
        qs = q * r_qt
        lo = jnp.zeros((L,), jnp.int32)
        hi = jnp.full((L,), N, jnp.int32)

        def step(_, lohi):
            lo, hi = lohi
            mid = lax.shift_right_logical(lo + hi, 1)
            tv = plsc.load_gather(t_ref, [mid])
            pred = (tv <= qs) if right else (tv < qs)
            return jnp.where(pred, mid + 1, lo), jnp.where(pred, hi, mid)

        c, _ = lax.fori_loop(0, BS_STEPS, step, (lo, hi))
        tprev = plsc.load_gather(t_ref, [jnp.maximum(c - 1, 0)])
        tprev = jnp.where(c > 0, tprev * r_tq, 0.0)
        qprev = _vgather(q, shift_idx)
        qprev = jnp.where(lanes == 0, carry_q, qprev)
        ivec = g * L + lanes
        delta = q - jnp.maximum(qprev, tprev)
        di = (ivec - jnp.minimum(c, N - 1)).astype(jnp.float32)
        return delta * di * di, _bcast_last(q)

    def group2(i, state):
        acc0, acc1, carry_q = state
        w0, carry_q = one_group(2 * i, carry_q)
        w1, carry_q = one_group(2 * i + 1, carry_q)
        return acc0 + w0, acc1 + w1, carry_q

    z = jnp.zeros((L,), jnp.float32)
    acc0, acc1, _ = lax.fori_loop(0, GROUPS // 2, group2, (z, z, z))
    return acc0 + acc1


def _sc_body(xt_hbm, yt_hbm, out_hbm, a0_v, b0_v, a1_v, b1_v, w_v, sem0, sem1):
    wid = lax.axis_index("s") * NC + lax.axis_index("c")
    lanes = lax.iota(jnp.int32, L)
    base = wid * RPW

    def start(r, a_v, b_v, sem):
        pltpu.async_copy(
            xt_hbm.at[pl.ds((base + r) * N, N)], a_v.at[pl.ds(0, N)], sem)
        pltpu.async_copy(
            yt_hbm.at[pl.ds((base + r) * N, N)], b_v.at[pl.ds(0, N)], sem)

    def wait(r, a_v, b_v, sem):
        pltpu.make_async_copy(
            xt_hbm.at[pl.ds((base + r) * N, N)], a_v.at[pl.ds(0, N)], sem
        ).wait()
        pltpu.make_async_copy(
            yt_hbm.at[pl.ds((base + r) * N, N)], b_v.at[pl.ds(0, N)], sem
        ).wait()

    def compute(r, a_v, b_v, wvec):
        ta_v = _cumsum_inplace(a_v)
        tb_v = _cumsum_inplace(b_v)
        inf = jnp.full((L,), jnp.inf, jnp.float32)
        a_v[pl.ds(N, L)] = inf
        b_v[pl.ds(N, L)] = inf
        r_ab = tb_v / ta_v
        r_ba = ta_v / tb_v
        acc = (_direction(a_v, b_v, r_ab, r_ba, right=False) / ta_v
               + _direction(b_v, a_v, r_ba, r_ab, right=True) / tb_v)
        w = jnp.sum(acc) * (DT * DT)
        return jnp.where(lanes == r, w, wvec)

    start(0, a0_v, b0_v, sem0)

    def row_pair(i, wvec):
        r0 = 2 * i
        r1 = 2 * i + 1
        start(r1, a1_v, b1_v, sem1)
        wait(r0, a0_v, b0_v, sem0)
        wvec = compute(r0, a0_v, b0_v, wvec)

        @pl.when(i < RPW // 2 - 1)
        def _():
            start(r0 + 2, a0_v, b0_v, sem0)

        wait(r1, a1_v, b1_v, sem1)
        return compute(r1, a1_v, b1_v, wvec)

    wvec = lax.fori_loop(0, RPW // 2, row_pair, jnp.zeros((L,), jnp.float32))
    w_v[...] = wvec
    pltpu.sync_copy(w_v, out_hbm.at[wid])


def _sum_body(p_ref, o_ref):
    o_ref[...] = jnp.full((1, 1), jnp.sum(p_ref[...]), jnp.float32)


@jax.jit
def kernel(x, y):
    nb, nt, nr, nc = x.shape
    x2 = x.reshape(nb, nt, nr * nc)
    y2 = y.reshape(nb, nt, nr * nc)

    xt, yt = pl.pallas_call(
        _prep_body,
        out_shape=[
            jax.ShapeDtypeStruct((nb, nr * nc, nt), jnp.float32),
            jax.ShapeDtypeStruct((nb, nr * nc, nt), jnp.float32),
        ],
    )(x2, y2)

    mesh = plsc.VectorSubcoreMesh(
        core_axis_name="c", subcore_axis_name="s", num_cores=NC, num_subcores=NS
    )
    partials = pl.kernel(
        _sc_body,
        out_type=jax.ShapeDtypeStruct((NW, L), jnp.float32),
        mesh=mesh,
        compiler_params=pltpu.CompilerParams(needs_layout_passes=False),
        scratch_types=[
            pltpu.VMEM((N + L,), jnp.float32),
            pltpu.VMEM((N + L,), jnp.float32),
            pltpu.VMEM((N + L,), jnp.float32),
            pltpu.VMEM((N + L,), jnp.float32),
            pltpu.VMEM((L,), jnp.float32),
            pltpu.SemaphoreType.DMA,
            pltpu.SemaphoreType.DMA,
        ],
    )(xt.reshape(ROWS * N), yt.reshape(ROWS * N))

    loss = pl.pallas_call(
        _sum_body,
        out_shape=jax.ShapeDtypeStruct((1, 1), jnp.float32),
    )(partials.reshape(4, 128))
    return loss[0, 0]
